# baseline (speedup 1.0000x reference)
import functools

import jax
import jax.numpy as jnp
from jax import lax
from jax.experimental import pallas as pl
from jax.experimental.pallas import tpu as pltpu
from jax.experimental.pallas import tpu_sc as plsc

B = 16384
D = 64
NU = 1000000
L = 16
NC = 2
NS = 16
NW = NC * NS

R = 31232
CW = 512
NCHW = R // CW
LO31 = (NW - 1) * R
NCH31 = (NU - LO31) // CW
TAIL_LO = LO31 + NCH31 * CW
TAIL_W = NU - TAIL_LO
DUMP = B
NR = 2
NBUF = 2

BPW = B // NW
CH = 128


def _extract_body(idx_hbm, tab_hbm, tail_hbm, vecs_hbm,
                  idxv, wlp, cl, chunkb, tailbuf, staging,
                  sem_c, sem_s):
    wid = lax.axis_index("s") * NC + lax.axis_index("c")
    lo = wid * R
    hi = jnp.where(wid == NW - 1, NU, lo + R)

    pltpu.sync_copy(idx_hbm, idxv)
    lane = lax.iota(jnp.int32, L)

    def scanv(v, off):
        u = idxv[pl.ds(v * L, L)]
        m = (u >= lo) & (u < hi)
        plsc.store_compressed(wlp.at[pl.ds(off, L)], v * L + lane, mask=m)
        return off + plsc.all_reduce_population_count(m)[0]

    n_w = lax.fori_loop(0, B // L, scanv, 0)

    def do_chunk(buf, clo, cw, ro):
        def cscan(g, off):
            valid = (g * L + lane) < n_w
            pv = wlp[pl.ds(g * L, L)] & (B - 1)
            uv = plsc.load_gather(idxv, [pv])
            m = valid & (uv >= clo) & (uv < clo + cw)
            packed = (uv - clo) | (pv << 16)
            plsc.store_compressed(cl.at[pl.ds(off, L)], packed, mask=m)
            return off + plsc.all_reduce_population_count(m)[0]

        n_c = lax.fori_loop(0, (n_w + L - 1) // L, cscan, 0)

        def egroup(g, ro):
            r, o = ro

            @pl.when(o >= NR)
            def _drain():
                pltpu.make_async_copy(
                    vecs_hbm.at[pl.ds(0, L), :], staging.at[0], sem_s).wait()

            o = jnp.where(o >= NR, o - 1, o)
            slot = lax.rem(r, NR)
            valid = (g * L + lane) < n_c
            packed = cl[pl.ds(g * L, L)]
            uloc = jnp.where(valid, packed & 0xFFFF, 0)
            posd = jnp.where(valid, lax.shift_right_logical(packed, 16),
                             DUMP + wid)
            for d in range(D):
                dsplat = jnp.full((L,), d, jnp.int32)
                val = plsc.load_gather(buf, [dsplat, uloc])
                plsc.store_scatter(staging.at[slot], [lane, dsplat], val)
            pltpu.async_copy(staging.at[slot], vecs_hbm.at[posd], sem_s)
            return (r + 1, o + 1)

        return lax.fori_loop(0, (n_c + L - 1) // L, egroup, ro)

    nch = jnp.where(wid == NW - 1, NCH31, NCHW)

    def fire_chunk(clo, p):
        for b in range(D // 8):
            pltpu.async_copy(
                tab_hbm.at[pl.ds(8 * b, 8), pl.ds(clo, CW)],
                chunkb.at[p, pl.ds(8 * b, 8), :], sem_c)

    for p in range(NBUF - 1):
        fire_chunk(lo + p * CW, p)

    def chunkloop(c, ro):
        pltpu.make_async_copy(
            tab_hbm.at[:, pl.ds(0, CW)], chunkb.at[0], sem_c).wait()

        @pl.when(c + (NBUF - 1) < nch)
        def _prefetch():
            fire_chunk(lo + (c + (NBUF - 1)) * CW,
                       lax.rem(c + (NBUF - 1), NBUF))

        return do_chunk(chunkb.at[lax.rem(c, NBUF)], lo + c * CW, CW, ro)

    ro = lax.fori_loop(0, nch, chunkloop, (0, 0))

    @pl.when(wid == NW - 1)
    def _tail():
        pltpu.sync_copy(tail_hbm, tailbuf)
        r, o = do_chunk(tailbuf, TAIL_LO, TAIL_W, ro)
        _ = lax.fori_loop(0, o, lambda i, c: _drain_one(vecs_hbm, staging,
                                                        sem_s, c), 0)

    @pl.when(wid != NW - 1)
    def _nodrain():
        _, o = ro
        _ = lax.fori_loop(0, o, lambda i, c: _drain_one(vecs_hbm, staging,
                                                        sem_s, c), 0)


def _drain_one(vecs_hbm, staging, sem_s, c):
    pltpu.make_async_copy(
        vecs_hbm.at[pl.ds(0, L), :], staging.at[0], sem_s).wait()
    return c


@jax.jit
def _extract(idx, tab_t, tail_t):
    mesh = plsc.VectorSubcoreMesh(
        core_axis_name="c", subcore_axis_name="s",
        num_cores=NC, num_subcores=NS)
    return pl.kernel(
        _extract_body,
        out_type=jax.ShapeDtypeStruct((B + NW, 2 * D), jnp.float32),
        mesh=mesh,
        compiler_params=pltpu.CompilerParams(needs_layout_passes=False),
        scratch_types=[
            pltpu.VMEM((B,), jnp.int32),
            pltpu.VMEM((B + L,), jnp.int32),
            pltpu.VMEM((B + L,), jnp.int32),
            pltpu.VMEM((NBUF, D, CW), jnp.float32),
            pltpu.VMEM((D, TAIL_W), jnp.float32),
            pltpu.VMEM((NR, L, 2 * D), jnp.float32),
            pltpu.SemaphoreType.DMA,
            pltpu.SemaphoreType.DMA,
        ],
    )(idx, tab_t, tail_t)


def _dot_body(uvecs_hbm, ivecs_hbm, out_hbm,
              ub, ib, partials, outv, sem):
    wid = lax.axis_index("s") * NC + lax.axis_index("c")
    base = wid * BPW
    lane = lax.iota(jnp.int32, L)

    for j in range(BPW // CH):
        cu = pltpu.async_copy(
            uvecs_hbm.at[pl.ds(base + j * CH, CH), :], ub, sem)
        ci = pltpu.async_copy(
            ivecs_hbm.at[pl.ds(base + j * CH, CH), :], ib, sem)
        cu.wait()
        ci.wait()

        def row(r, carry):
            acc = ub[r, pl.ds(0, L)] * ib[r, pl.ds(0, L)]
            for g in range(1, D // L):
                acc = acc + ub[r, pl.ds(g * L, L)] * ib[r, pl.ds(g * L, L)]
            partials[pl.ds(r * L, L)] = acc
            return carry

        lax.fori_loop(0, CH, row, 0)

        def group(g, carry):
            idx = g * (L * L) + lane * L
            acc = plsc.load_gather(partials, [idx])
            for c in range(1, L):
                acc = acc + plsc.load_gather(partials, [idx + c])
            outv[pl.ds(j * CH + g * L, L)] = acc
            return carry

        lax.fori_loop(0, CH // L, group, 0)

    pltpu.sync_copy(outv, out_hbm.at[pl.ds(base, BPW)])


@jax.jit
def _dot(uvecs, ivecs):
    mesh = plsc.VectorSubcoreMesh(
        core_axis_name="c", subcore_axis_name="s",
        num_cores=NC, num_subcores=NS)
    return pl.kernel(
        _dot_body,
        out_type=jax.ShapeDtypeStruct((B,), jnp.float32),
        mesh=mesh,
        compiler_params=pltpu.CompilerParams(needs_layout_passes=False),
        scratch_types=[
            pltpu.VMEM((CH, 2 * D), jnp.float32),
            pltpu.VMEM((CH, 2 * D), jnp.float32),
            pltpu.VMEM((CH * L,), jnp.float32),
            pltpu.VMEM((BPW,), jnp.float32),
            pltpu.SemaphoreType.DMA,
        ],
    )(uvecs, ivecs)


def kernel(user, item, user_emb_w, item_emb_w):
    ut = user_emb_w.T
    it = item_emb_w.T
    uvecs = _extract(user.astype(jnp.int32), ut, ut[:, TAIL_LO:])
    ivecs = _extract(item.astype(jnp.int32), it, it[:, TAIL_LO:])
    return _dot(uvecs, ivecs)


# speedup vs baseline: 1.0020x; 1.0020x over previous
import functools

import jax
import jax.numpy as jnp
from jax import lax
from jax.experimental import pallas as pl
from jax.experimental.pallas import tpu as pltpu
from jax.experimental.pallas import tpu_sc as plsc

B = 16384
D = 64
NU = 1000000
L = 16
NC = 2
NS = 16
NW = NC * NS

R = 31232
CW = 512
NCHW = R // CW
LO31 = (NW - 1) * R
NCH31 = (NU - LO31) // CW
TAIL_LO = LO31 + NCH31 * CW
TAIL_W = NU - TAIL_LO
DUMP = B
NR = 2
NBUF = 2

BPW = B // NW
CH = 128


def _extract_body(idx_hbm, tab_hbm, tail_hbm, vecs_hbm,
                  idxv, wlp, cl, chunkb, tailbuf, staging,
                  sem_c, sem_s):
    wid = lax.axis_index("s") * NC + lax.axis_index("c")
    lo = wid * R
    hi = jnp.where(wid == NW - 1, NU, lo + R)

    pltpu.sync_copy(idx_hbm, idxv)
    lane = lax.iota(jnp.int32, L)

    def scanv(v, off):
        u = idxv[pl.ds(v * L, L)]
        m = (u >= lo) & (u < hi)
        plsc.store_compressed(wlp.at[pl.ds(off, L)], v * L + lane, mask=m)
        return off + plsc.all_reduce_population_count(m)[0]

    n_w = lax.fori_loop(0, B // L, scanv, 0, unroll=4)

    def do_chunk(buf, clo, cw, ro):
        def cscan(g, off):
            for k in range(4):
                base = g * (4 * L) + k * L
                valid = (base + lane) < n_w
                pv = wlp[pl.ds(base, L)] & (B - 1)
                uv = plsc.load_gather(idxv, [pv])
                m = valid & (uv >= clo) & (uv < clo + cw)
                packed = (uv - clo) | (pv << 16)
                plsc.store_compressed(cl.at[pl.ds(off, L)], packed, mask=m)
                off = off + plsc.all_reduce_population_count(m)[0]
            return off

        n_c = lax.fori_loop(0, (n_w + 4 * L - 1) // (4 * L), cscan, 0)

        def egroup(g, ro):
            r, o = ro

            @pl.when(o >= NR)
            def _drain():
                pltpu.make_async_copy(
                    vecs_hbm.at[pl.ds(0, L), :], staging.at[0], sem_s).wait()

            o = jnp.where(o >= NR, o - 1, o)
            slot = lax.rem(r, NR)
            valid = (g * L + lane) < n_c
            packed = cl[pl.ds(g * L, L)]
            uloc = jnp.where(valid, packed & 0xFFFF, 0)
            posd = jnp.where(valid, lax.shift_right_logical(packed, 16),
                             DUMP + wid)
            for d in range(D):
                dsplat = jnp.full((L,), d, jnp.int32)
                val = plsc.load_gather(buf, [dsplat, uloc])
                plsc.store_scatter(staging.at[slot], [lane, dsplat], val)
            pltpu.async_copy(staging.at[slot], vecs_hbm.at[posd], sem_s)
            return (r + 1, o + 1)

        return lax.fori_loop(0, (n_c + L - 1) // L, egroup, ro)

    nch = jnp.where(wid == NW - 1, NCH31, NCHW)

    def fire_chunk(clo, p):
        for b in range(D // 8):
            pltpu.async_copy(
                tab_hbm.at[pl.ds(8 * b, 8), pl.ds(clo, CW)],
                chunkb.at[p, pl.ds(8 * b, 8), :], sem_c)

    for p in range(NBUF - 1):
        fire_chunk(lo + p * CW, p)

    def chunkloop(c, ro):
        pltpu.make_async_copy(
            tab_hbm.at[:, pl.ds(0, CW)], chunkb.at[0], sem_c).wait()

        @pl.when(c + (NBUF - 1) < nch)
        def _prefetch():
            fire_chunk(lo + (c + (NBUF - 1)) * CW,
                       lax.rem(c + (NBUF - 1), NBUF))

        return do_chunk(chunkb.at[lax.rem(c, NBUF)], lo + c * CW, CW, ro)

    ro = lax.fori_loop(0, nch, chunkloop, (0, 0))

    @pl.when(wid == NW - 1)
    def _tail():
        pltpu.sync_copy(tail_hbm, tailbuf)
        r, o = do_chunk(tailbuf, TAIL_LO, TAIL_W, ro)
        _ = lax.fori_loop(0, o, lambda i, c: _drain_one(vecs_hbm, staging,
                                                        sem_s, c), 0)

    @pl.when(wid != NW - 1)
    def _nodrain():
        _, o = ro
        _ = lax.fori_loop(0, o, lambda i, c: _drain_one(vecs_hbm, staging,
                                                        sem_s, c), 0)


def _drain_one(vecs_hbm, staging, sem_s, c):
    pltpu.make_async_copy(
        vecs_hbm.at[pl.ds(0, L), :], staging.at[0], sem_s).wait()
    return c


@jax.jit
def _extract(idx, tab_t, tail_t):
    mesh = plsc.VectorSubcoreMesh(
        core_axis_name="c", subcore_axis_name="s",
        num_cores=NC, num_subcores=NS)
    return pl.kernel(
        _extract_body,
        out_type=jax.ShapeDtypeStruct((B + NW, 2 * D), jnp.float32),
        mesh=mesh,
        compiler_params=pltpu.CompilerParams(needs_layout_passes=False),
        scratch_types=[
            pltpu.VMEM((B,), jnp.int32),
            pltpu.VMEM((B + L,), jnp.int32),
            pltpu.VMEM((B + L,), jnp.int32),
            pltpu.VMEM((NBUF, D, CW), jnp.float32),
            pltpu.VMEM((D, TAIL_W), jnp.float32),
            pltpu.VMEM((NR, L, 2 * D), jnp.float32),
            pltpu.SemaphoreType.DMA,
            pltpu.SemaphoreType.DMA,
        ],
    )(idx, tab_t, tail_t)


def _dot_body(uvecs_hbm, ivecs_hbm, out_hbm,
              ub, ib, partials, outv, sem):
    wid = lax.axis_index("s") * NC + lax.axis_index("c")
    base = wid * BPW
    lane = lax.iota(jnp.int32, L)

    for j in range(BPW // CH):
        cu = pltpu.async_copy(
            uvecs_hbm.at[pl.ds(base + j * CH, CH), :], ub, sem)
        ci = pltpu.async_copy(
            ivecs_hbm.at[pl.ds(base + j * CH, CH), :], ib, sem)
        cu.wait()
        ci.wait()

        def row(r, carry):
            acc = ub[r, pl.ds(0, L)] * ib[r, pl.ds(0, L)]
            for g in range(1, D // L):
                acc = acc + ub[r, pl.ds(g * L, L)] * ib[r, pl.ds(g * L, L)]
            partials[pl.ds(r * L, L)] = acc
            return carry

        lax.fori_loop(0, CH, row, 0)

        def group(g, carry):
            idx = g * (L * L) + lane * L
            acc = plsc.load_gather(partials, [idx])
            for c in range(1, L):
                acc = acc + plsc.load_gather(partials, [idx + c])
            outv[pl.ds(j * CH + g * L, L)] = acc
            return carry

        lax.fori_loop(0, CH // L, group, 0)

    pltpu.sync_copy(outv, out_hbm.at[pl.ds(base, BPW)])


@jax.jit
def _dot(uvecs, ivecs):
    mesh = plsc.VectorSubcoreMesh(
        core_axis_name="c", subcore_axis_name="s",
        num_cores=NC, num_subcores=NS)
    return pl.kernel(
        _dot_body,
        out_type=jax.ShapeDtypeStruct((B,), jnp.float32),
        mesh=mesh,
        compiler_params=pltpu.CompilerParams(needs_layout_passes=False),
        scratch_types=[
            pltpu.VMEM((CH, 2 * D), jnp.float32),
            pltpu.VMEM((CH, 2 * D), jnp.float32),
            pltpu.VMEM((CH * L,), jnp.float32),
            pltpu.VMEM((BPW,), jnp.float32),
            pltpu.SemaphoreType.DMA,
        ],
    )(uvecs, ivecs)


def kernel(user, item, user_emb_w, item_emb_w):
    ut = user_emb_w.T
    it = item_emb_w.T
    uvecs = _extract(user.astype(jnp.int32), ut, ut[:, TAIL_LO:])
    ivecs = _extract(item.astype(jnp.int32), it, it[:, TAIL_LO:])
    return _dot(uvecs, ivecs)


# speedup vs baseline: 1.0573x; 1.0551x over previous
import functools

import jax
import jax.numpy as jnp
from jax import lax
from jax.experimental import pallas as pl
from jax.experimental.pallas import tpu as pltpu
from jax.experimental.pallas import tpu_sc as plsc

B = 16384
D = 64
NU = 1000000
L = 16
NC = 2
NS = 16
NW = NC * NS

R = 31232
CW = 512
NCHW = R // CW
LO31 = (NW - 1) * R
NCH31 = (NU - LO31) // CW
TAIL_LO = LO31 + NCH31 * CW
TAIL_W = NU - TAIL_LO
DUMP = B
NR = 2
NBUF = 2

BPW = B // NW
CH = 128


def _extract_body(user_hbm, item_hbm, ut_hbm, it_hbm, utail_hbm, itail_hbm,
                  uvecs_hbm, ivecs_hbm,
                  idxv, wlp, cl, chunkb, tailbuf, staging,
                  sem_c, sem_s):
    wid = lax.axis_index("s") * NC + lax.axis_index("c")
    lo = wid * R
    hi = jnp.where(wid == NW - 1, NU, lo + R)
    for (idx_hbm, tab_hbm, tail_hbm, vecs_hbm) in (
            (user_hbm, ut_hbm, utail_hbm, uvecs_hbm),
            (item_hbm, it_hbm, itail_hbm, ivecs_hbm)):
        _one_table(idx_hbm, tab_hbm, tail_hbm, vecs_hbm,
                   idxv, wlp, cl, chunkb, tailbuf, staging,
                   sem_c, sem_s, wid, lo, hi)


def _one_table(idx_hbm, tab_hbm, tail_hbm, vecs_hbm,
               idxv, wlp, cl, chunkb, tailbuf, staging,
               sem_c, sem_s, wid, lo, hi):
    pltpu.sync_copy(idx_hbm, idxv)
    lane = lax.iota(jnp.int32, L)

    def scanv(v, off):
        u = idxv[pl.ds(v * L, L)]
        m = (u >= lo) & (u < hi)
        plsc.store_compressed(wlp.at[pl.ds(off, L)], v * L + lane, mask=m)
        return off + plsc.all_reduce_population_count(m)[0]

    n_w = lax.fori_loop(0, B // L, scanv, 0, unroll=4)

    def do_chunk(buf, clo, cw, ro):
        def cscan(g, off):
            for k in range(4):
                base = g * (4 * L) + k * L
                valid = (base + lane) < n_w
                pv = wlp[pl.ds(base, L)] & (B - 1)
                uv = plsc.load_gather(idxv, [pv])
                m = valid & (uv >= clo) & (uv < clo + cw)
                packed = (uv - clo) | (pv << 16)
                plsc.store_compressed(cl.at[pl.ds(off, L)], packed, mask=m)
                off = off + plsc.all_reduce_population_count(m)[0]
            return off

        n_c = lax.fori_loop(0, (n_w + 4 * L - 1) // (4 * L), cscan, 0)

        def egroup(g, ro):
            r, o = ro

            @pl.when(o >= NR)
            def _drain():
                pltpu.make_async_copy(
                    vecs_hbm.at[pl.ds(0, L), :], staging.at[0], sem_s).wait()

            o = jnp.where(o >= NR, o - 1, o)
            slot = lax.rem(r, NR)
            valid = (g * L + lane) < n_c
            packed = cl[pl.ds(g * L, L)]
            uloc = jnp.where(valid, packed & 0xFFFF, 0)
            posd = jnp.where(valid, lax.shift_right_logical(packed, 16),
                             DUMP + wid)
            for d in range(D):
                dsplat = jnp.full((L,), d, jnp.int32)
                val = plsc.load_gather(buf, [dsplat, uloc])
                plsc.store_scatter(staging.at[slot], [lane, dsplat], val)
            pltpu.async_copy(staging.at[slot], vecs_hbm.at[posd], sem_s)
            return (r + 1, o + 1)

        return lax.fori_loop(0, (n_c + L - 1) // L, egroup, ro)

    nch = jnp.where(wid == NW - 1, NCH31, NCHW)

    def fire_chunk(clo, p):
        for b in range(D // 8):
            pltpu.async_copy(
                tab_hbm.at[pl.ds(8 * b, 8), pl.ds(clo, CW)],
                chunkb.at[p, pl.ds(8 * b, 8), :], sem_c)

    for p in range(NBUF - 1):
        fire_chunk(lo + p * CW, p)

    def chunkloop(c, ro):
        pltpu.make_async_copy(
            tab_hbm.at[:, pl.ds(0, CW)], chunkb.at[0], sem_c).wait()

        @pl.when(c + (NBUF - 1) < nch)
        def _prefetch():
            fire_chunk(lo + (c + (NBUF - 1)) * CW,
                       lax.rem(c + (NBUF - 1), NBUF))

        return do_chunk(chunkb.at[lax.rem(c, NBUF)], lo + c * CW, CW, ro)

    ro = lax.fori_loop(0, nch, chunkloop, (0, 0))

    @pl.when(wid == NW - 1)
    def _tail():
        pltpu.sync_copy(tail_hbm, tailbuf)
        r, o = do_chunk(tailbuf, TAIL_LO, TAIL_W, ro)
        _ = lax.fori_loop(0, o, lambda i, c: _drain_one(vecs_hbm, staging,
                                                        sem_s, c), 0)

    @pl.when(wid != NW - 1)
    def _nodrain():
        _, o = ro
        _ = lax.fori_loop(0, o, lambda i, c: _drain_one(vecs_hbm, staging,
                                                        sem_s, c), 0)


def _drain_one(vecs_hbm, staging, sem_s, c):
    pltpu.make_async_copy(
        vecs_hbm.at[pl.ds(0, L), :], staging.at[0], sem_s).wait()
    return c


@jax.jit
def _extract(user, item, ut, it, utail, itail):
    mesh = plsc.VectorSubcoreMesh(
        core_axis_name="c", subcore_axis_name="s",
        num_cores=NC, num_subcores=NS)
    return pl.kernel(
        _extract_body,
        out_type=(jax.ShapeDtypeStruct((B + NW, 2 * D), jnp.float32),
                  jax.ShapeDtypeStruct((B + NW, 2 * D), jnp.float32)),
        mesh=mesh,
        compiler_params=pltpu.CompilerParams(needs_layout_passes=False),
        scratch_types=[
            pltpu.VMEM((B,), jnp.int32),
            pltpu.VMEM((B + L,), jnp.int32),
            pltpu.VMEM((B + L,), jnp.int32),
            pltpu.VMEM((NBUF, D, CW), jnp.float32),
            pltpu.VMEM((D, TAIL_W), jnp.float32),
            pltpu.VMEM((NR, L, 2 * D), jnp.float32),
            pltpu.SemaphoreType.DMA,
            pltpu.SemaphoreType.DMA,
        ],
    )(user, item, ut, it, utail, itail)


def _dot_body(uvecs_hbm, ivecs_hbm, out_hbm,
              ub, ib, partials, outv, sem):
    wid = lax.axis_index("s") * NC + lax.axis_index("c")
    base = wid * BPW
    lane = lax.iota(jnp.int32, L)

    for j in range(BPW // CH):
        cu = pltpu.async_copy(
            uvecs_hbm.at[pl.ds(base + j * CH, CH), :], ub, sem)
        ci = pltpu.async_copy(
            ivecs_hbm.at[pl.ds(base + j * CH, CH), :], ib, sem)
        cu.wait()
        ci.wait()

        def row(r, carry):
            acc = ub[r, pl.ds(0, L)] * ib[r, pl.ds(0, L)]
            for g in range(1, D // L):
                acc = acc + ub[r, pl.ds(g * L, L)] * ib[r, pl.ds(g * L, L)]
            partials[pl.ds(r * L, L)] = acc
            return carry

        lax.fori_loop(0, CH, row, 0)

        def group(g, carry):
            idx = g * (L * L) + lane * L
            acc = plsc.load_gather(partials, [idx])
            for c in range(1, L):
                acc = acc + plsc.load_gather(partials, [idx + c])
            outv[pl.ds(j * CH + g * L, L)] = acc
            return carry

        lax.fori_loop(0, CH // L, group, 0)

    pltpu.sync_copy(outv, out_hbm.at[pl.ds(base, BPW)])


@jax.jit
def _dot(uvecs, ivecs):
    mesh = plsc.VectorSubcoreMesh(
        core_axis_name="c", subcore_axis_name="s",
        num_cores=NC, num_subcores=NS)
    return pl.kernel(
        _dot_body,
        out_type=jax.ShapeDtypeStruct((B,), jnp.float32),
        mesh=mesh,
        compiler_params=pltpu.CompilerParams(needs_layout_passes=False),
        scratch_types=[
            pltpu.VMEM((CH, 2 * D), jnp.float32),
            pltpu.VMEM((CH, 2 * D), jnp.float32),
            pltpu.VMEM((CH * L,), jnp.float32),
            pltpu.VMEM((BPW,), jnp.float32),
            pltpu.SemaphoreType.DMA,
        ],
    )(uvecs, ivecs)


def kernel(user, item, user_emb_w, item_emb_w):
    ut = user_emb_w.T
    it = item_emb_w.T
    uvecs, ivecs = _extract(user.astype(jnp.int32), item.astype(jnp.int32),
                            ut, it, ut[:, TAIL_LO:], it[:, TAIL_LO:])
    return _dot(uvecs, ivecs)


# speedup vs baseline: 1.0701x; 1.0122x over previous
import functools

import jax
import jax.numpy as jnp
from jax import lax
from jax.experimental import pallas as pl
from jax.experimental.pallas import tpu as pltpu
from jax.experimental.pallas import tpu_sc as plsc

B = 16384
D = 64
NU = 1000000
L = 16
NC = 2
NS = 16
NW = NC * NS

R = 31232
CW = 512
NCHW = R // CW
LO31 = (NW - 1) * R
NCH31 = (NU - LO31) // CW
TAIL_LO = LO31 + NCH31 * CW
TAIL_W = NU - TAIL_LO
DUMP = B
NR = 2
NBUF = 2

BPW = B // NW
CH = 128


def _extract_body(user_hbm, item_hbm, ut_hbm, it_hbm, utail_hbm, itail_hbm,
                  uvecs_hbm, ivecs_hbm,
                  idxv, wlp, cl, chunkb, tailbuf, staging,
                  sem_c, sem_s):
    wid = lax.axis_index("s") * NC + lax.axis_index("c")
    lo = wid * R
    hi = jnp.where(wid == NW - 1, NU, lo + R)
    for (idx_hbm, tab_hbm, tail_hbm, vecs_hbm) in (
            (user_hbm, ut_hbm, utail_hbm, uvecs_hbm),
            (item_hbm, it_hbm, itail_hbm, ivecs_hbm)):
        _one_table(idx_hbm, tab_hbm, tail_hbm, vecs_hbm,
                   idxv, wlp, cl, chunkb, tailbuf, staging,
                   sem_c, sem_s, wid, lo, hi)


def _one_table(idx_hbm, tab_hbm, tail_hbm, vecs_hbm,
               idxv, wlp, cl, chunkb, tailbuf, staging,
               sem_c, sem_s, wid, lo, hi):
    pltpu.sync_copy(idx_hbm, idxv)
    lane = lax.iota(jnp.int32, L)

    def scanv(v, off):
        u = idxv[pl.ds(v * L, L)]
        m = (u >= lo) & (u < hi)
        plsc.store_compressed(wlp.at[pl.ds(off, L)], v * L + lane, mask=m)
        return off + plsc.all_reduce_population_count(m)[0]

    n_w = lax.fori_loop(0, B // L, scanv, 0, unroll=4)

    def do_chunk(buf, clo, cw, ro):
        def cscan(g, off):
            for k in range(4):
                base = g * (4 * L) + k * L
                valid = (base + lane) < n_w
                pv = wlp[pl.ds(base, L)] & (B - 1)
                uv = plsc.load_gather(idxv, [pv])
                m = valid & (uv >= clo) & (uv < clo + cw)
                packed = (uv - clo) | (pv << 16)
                plsc.store_compressed(cl.at[pl.ds(off, L)], packed, mask=m)
                off = off + plsc.all_reduce_population_count(m)[0]
            return off

        n_c = lax.fori_loop(0, (n_w + 4 * L - 1) // (4 * L), cscan, 0)

        def egroup(g, ro):
            r, o = ro

            @pl.when(o >= NR)
            def _drain():
                pltpu.make_async_copy(
                    vecs_hbm.at[pl.ds(0, L), :], staging.at[0], sem_s).wait()

            o = jnp.where(o >= NR, o - 1, o)
            slot = lax.rem(r, NR)
            valid = (g * L + lane) < n_c
            packed = cl[pl.ds(g * L, L)]
            uloc = jnp.where(valid, packed & 0xFFFF, 0)
            posd = jnp.where(valid, lax.shift_right_logical(packed, 16),
                             DUMP + wid)
            for d in range(D):
                dsplat = jnp.full((L,), d, jnp.int32)
                val = plsc.load_gather(buf, [dsplat, uloc])
                plsc.store_scatter(staging.at[slot], [lane, dsplat], val)
            pltpu.async_copy(staging.at[slot], vecs_hbm.at[posd], sem_s)
            return (r + 1, o + 1)

        return lax.fori_loop(0, (n_c + L - 1) // L, egroup, ro)

    nch = jnp.where(wid == NW - 1, NCH31, NCHW)

    def fire_chunk(clo, p):
        for b in range(D // 8):
            pltpu.async_copy(
                tab_hbm.at[pl.ds(8 * b, 8), pl.ds(clo, CW)],
                chunkb.at[p, pl.ds(8 * b, 8), :], sem_c)

    for p in range(NBUF - 1):
        fire_chunk(lo + p * CW, p)

    def chunkloop(c, ro):
        pltpu.make_async_copy(
            tab_hbm.at[:, pl.ds(0, CW)], chunkb.at[0], sem_c).wait()

        @pl.when(c + (NBUF - 1) < nch)
        def _prefetch():
            fire_chunk(lo + (c + (NBUF - 1)) * CW,
                       lax.rem(c + (NBUF - 1), NBUF))

        return do_chunk(chunkb.at[lax.rem(c, NBUF)], lo + c * CW, CW, ro)

    ro = lax.fori_loop(0, nch, chunkloop, (0, 0))

    @pl.when(wid == NW - 1)
    def _tail():
        pltpu.sync_copy(tail_hbm, tailbuf)
        r, o = do_chunk(tailbuf, TAIL_LO, TAIL_W, ro)
        _ = lax.fori_loop(0, o, lambda i, c: _drain_one(vecs_hbm, staging,
                                                        sem_s, c), 0)

    @pl.when(wid != NW - 1)
    def _nodrain():
        _, o = ro
        _ = lax.fori_loop(0, o, lambda i, c: _drain_one(vecs_hbm, staging,
                                                        sem_s, c), 0)


def _drain_one(vecs_hbm, staging, sem_s, c):
    pltpu.make_async_copy(
        vecs_hbm.at[pl.ds(0, L), :], staging.at[0], sem_s).wait()
    return c


@jax.jit
def _extract(user, item, ut, it, utail, itail):
    mesh = plsc.VectorSubcoreMesh(
        core_axis_name="c", subcore_axis_name="s",
        num_cores=NC, num_subcores=NS)
    return pl.kernel(
        _extract_body,
        out_type=(jax.ShapeDtypeStruct((B + NW, 2 * D), jnp.float32),
                  jax.ShapeDtypeStruct((B + NW, 2 * D), jnp.float32)),
        mesh=mesh,
        compiler_params=pltpu.CompilerParams(needs_layout_passes=False),
        scratch_types=[
            pltpu.VMEM((B,), jnp.int32),
            pltpu.VMEM((B + L,), jnp.int32),
            pltpu.VMEM((B + L,), jnp.int32),
            pltpu.VMEM((NBUF, D, CW), jnp.float32),
            pltpu.VMEM((D, TAIL_W), jnp.float32),
            pltpu.VMEM((NR, L, 2 * D), jnp.float32),
            pltpu.SemaphoreType.DMA,
            pltpu.SemaphoreType.DMA,
        ],
    )(user, item, ut, it, utail, itail)


def _dot_body(uvecs_hbm, ivecs_hbm, out_hbm,
              ub, ib, partials, outv, sem):
    wid = lax.axis_index("s") * NC + lax.axis_index("c")
    base = wid * BPW
    lane = lax.iota(jnp.int32, L)
    NJ = BPW // CH

    def fire(j, p):
        pltpu.async_copy(
            uvecs_hbm.at[pl.ds(base + j * CH, CH), :], ub.at[p], sem)
        pltpu.async_copy(
            ivecs_hbm.at[pl.ds(base + j * CH, CH), :], ib.at[p], sem)

    fire(0, 0)
    for j in range(NJ):
        pltpu.make_async_copy(
            uvecs_hbm.at[pl.ds(0, CH), :], ub.at[0], sem).wait()
        pltpu.make_async_copy(
            ivecs_hbm.at[pl.ds(0, CH), :], ib.at[0], sem).wait()
        if j + 1 < NJ:
            fire(j + 1, (j + 1) % 2)
        ubp = ub.at[j % 2]
        ibp = ib.at[j % 2]

        def row(r, carry):
            acc = ubp[r, pl.ds(0, L)] * ibp[r, pl.ds(0, L)]
            for g in range(1, D // L):
                acc = acc + ubp[r, pl.ds(g * L, L)] * ibp[r, pl.ds(g * L, L)]
            partials[pl.ds(r * L, L)] = acc
            return carry

        lax.fori_loop(0, CH, row, 0)

        def group(g, carry):
            idx = g * (L * L) + lane * L
            acc = plsc.load_gather(partials, [idx])
            for c in range(1, L):
                acc = acc + plsc.load_gather(partials, [idx + c])
            outv[pl.ds(j * CH + g * L, L)] = acc
            return carry

        lax.fori_loop(0, CH // L, group, 0)

    pltpu.sync_copy(outv, out_hbm.at[pl.ds(base, BPW)])


@jax.jit
def _dot(uvecs, ivecs):
    mesh = plsc.VectorSubcoreMesh(
        core_axis_name="c", subcore_axis_name="s",
        num_cores=NC, num_subcores=NS)
    return pl.kernel(
        _dot_body,
        out_type=jax.ShapeDtypeStruct((B,), jnp.float32),
        mesh=mesh,
        compiler_params=pltpu.CompilerParams(needs_layout_passes=False),
        scratch_types=[
            pltpu.VMEM((2, CH, 2 * D), jnp.float32),
            pltpu.VMEM((2, CH, 2 * D), jnp.float32),
            pltpu.VMEM((CH * L,), jnp.float32),
            pltpu.VMEM((BPW,), jnp.float32),
            pltpu.SemaphoreType.DMA,
        ],
    )(uvecs, ivecs)


def kernel(user, item, user_emb_w, item_emb_w):
    ut = user_emb_w.T
    it = item_emb_w.T
    uvecs, ivecs = _extract(user.astype(jnp.int32), item.astype(jnp.int32),
                            ut, it, ut[:, TAIL_LO:], it[:, TAIL_LO:])
    return _dot(uvecs, ivecs)
